# Initial kernel scaffold; baseline (speedup 1.0000x reference)
import functools

import jax
import jax.numpy as jnp
from jax import lax
from jax.experimental import pallas as pl
from jax.experimental.pallas import tpu as pltpu
from jax.experimental.pallas import tpu_sc as plsc

_LANES = 16
_NUM_SUBCORES = 16


@functools.lru_cache(maxsize=None)
def _build(n_nodes, n_pins):
    pins_per_tile = n_pins // _NUM_SUBCORES
    chunk = 10000
    assert pins_per_tile % chunk == 0 and chunk % _LANES == 0

    mesh = plsc.VectorSubcoreMesh(core_axis_name="c", subcore_axis_name="s")

    @functools.partial(
        pl.kernel,
        mesh=mesh,
        out_type=jax.ShapeDtypeStruct((2 * n_pins,), jnp.float32),
        scratch_types=[
            pltpu.VMEM((n_nodes,), jnp.float32),
            pltpu.VMEM((chunk,), jnp.int32),
            pltpu.VMEM((chunk,), jnp.float32),
        ],
    )
    def pin_pos(pos_hbm, offx_hbm, offy_hbm, p2n_hbm, out_hbm,
                table_v, idx_v, data_v):
        coord = lax.axis_index("c")
        sid = lax.axis_index("s")

        pltpu.sync_copy(pos_hbm.at[pl.ds(coord * n_nodes, n_nodes)], table_v)

        base0 = sid * pins_per_tile

        def chunk_body(j, _):
            base = base0 + j * chunk
            pltpu.sync_copy(p2n_hbm.at[pl.ds(base, chunk)], idx_v)

            @pl.when(coord == 0)
            def _():
                pltpu.sync_copy(offx_hbm.at[pl.ds(base, chunk)], data_v)

            @pl.when(coord == 1)
            def _():
                pltpu.sync_copy(offy_hbm.at[pl.ds(base, chunk)], data_v)

            def vec_body(k, _):
                sl = pl.ds(k * _LANES, _LANES)
                gathered = plsc.load_gather(table_v, [idx_v[sl]])
                data_v[sl] = data_v[sl] + gathered
                return 0

            lax.fori_loop(0, chunk // _LANES, vec_body, 0, unroll=4)

            pltpu.sync_copy(data_v, out_hbm.at[pl.ds(coord * n_pins + base, chunk)])
            return 0

        lax.fori_loop(0, pins_per_tile // chunk, chunk_body, 0)

    return pin_pos


def kernel(pos, pin_offset_x, pin_offset_y, pin2node_map, flat_node2pin_map,
           flat_node2pin_start_map, num_physical_nodes):
    n_pins = pin2node_map.shape[0]
    n_nodes = pos.shape[0] // 2
    return _build(n_nodes, n_pins)(pos, pin_offset_x, pin_offset_y, pin2node_map)


# speedup vs baseline: 286.7803x; 286.7803x over previous
import functools

import jax
import jax.numpy as jnp
from jax import lax
from jax.experimental import pallas as pl
from jax.experimental.pallas import tpu as pltpu
from jax.experimental.pallas import tpu_sc as plsc

_LANES = 16
_NUM_WORKERS = 32


@functools.lru_cache(maxsize=None)
def _build(n_nodes, n_pins):
    pins_per_tile = n_pins // _NUM_WORKERS
    chunk = 10000
    assert pins_per_tile % chunk == 0 and chunk % _LANES == 0

    mesh = plsc.VectorSubcoreMesh(core_axis_name="c", subcore_axis_name="s")

    @functools.partial(
        pl.kernel,
        mesh=mesh,
        out_type=jax.ShapeDtypeStruct((2 * n_pins,), jnp.float32),
        compiler_params=pltpu.CompilerParams(needs_layout_passes=False),
        scratch_types=[
            pltpu.VMEM((n_nodes,), jnp.float32),
            pltpu.VMEM((chunk,), jnp.int32),
            pltpu.VMEM((chunk,), jnp.float32),
        ],
    )
    def pin_pos(pos_hbm, offx_hbm, offy_hbm, p2n_hbm, out_hbm,
                table_v, idx_v, data_v):
        wid = lax.axis_index("s") * 2 + lax.axis_index("c")
        base0 = wid * pins_per_tile

        def one_pass(table_base, off_hbm, out_base):
            pltpu.sync_copy(pos_hbm.at[pl.ds(table_base, n_nodes)], table_v)

            def chunk_body(j, _):
                base = base0 + j * chunk
                pltpu.sync_copy(p2n_hbm.at[pl.ds(base, chunk)], idx_v)
                pltpu.sync_copy(off_hbm.at[pl.ds(base, chunk)], data_v)

                def vec_body(k, _):
                    sl = pl.ds(k * _LANES, _LANES)
                    gathered = plsc.load_gather(table_v, [idx_v[sl]])
                    data_v[sl] = data_v[sl] + gathered
                    return 0

                lax.fori_loop(0, chunk // _LANES, vec_body, 0, unroll=4)

                pltpu.sync_copy(data_v, out_hbm.at[pl.ds(out_base + base, chunk)])
                return 0

            lax.fori_loop(0, pins_per_tile // chunk, chunk_body, 0)

        one_pass(0, offx_hbm, 0)
        one_pass(n_nodes, offy_hbm, n_pins)

    return pin_pos


def kernel(pos, pin_offset_x, pin_offset_y, pin2node_map, flat_node2pin_map,
           flat_node2pin_start_map, num_physical_nodes):
    n_pins = pin2node_map.shape[0]
    n_nodes = pos.shape[0] // 2
    return _build(n_nodes, n_pins)(pos, pin_offset_x, pin_offset_y, pin2node_map)


# speedup vs baseline: 383.4322x; 1.3370x over previous
import functools

import jax
import jax.numpy as jnp
from jax import lax
from jax.experimental import pallas as pl
from jax.experimental.pallas import tpu as pltpu
from jax.experimental.pallas import tpu_sc as plsc

_LANES = 16
_NUM_WORKERS = 32


@functools.lru_cache(maxsize=None)
def _build(n_nodes, n_pins):
    pins_per_tile = n_pins // _NUM_WORKERS
    chunk = 2000
    num_chunks = pins_per_tile // chunk
    assert pins_per_tile % chunk == 0 and chunk % _LANES == 0
    assert num_chunks % 2 == 0
    half = num_chunks // 2

    mesh = plsc.VectorSubcoreMesh(core_axis_name="c", subcore_axis_name="s")

    @functools.partial(
        pl.kernel,
        mesh=mesh,
        out_type=jax.ShapeDtypeStruct((2 * n_pins,), jnp.float32),
        compiler_params=pltpu.CompilerParams(needs_layout_passes=False),
        scratch_types=[
            pltpu.VMEM((n_nodes,), jnp.float32),
            pltpu.VMEM((chunk,), jnp.int32),
            pltpu.VMEM((chunk,), jnp.int32),
            pltpu.VMEM((chunk,), jnp.float32),
            pltpu.VMEM((chunk,), jnp.float32),
            pltpu.VMEM((chunk,), jnp.float32),
            pltpu.VMEM((chunk,), jnp.float32),
            pltpu.SemaphoreType.DMA,
            pltpu.SemaphoreType.DMA,
            pltpu.SemaphoreType.DMA,
            pltpu.SemaphoreType.DMA,
            pltpu.SemaphoreType.DMA,
        ],
    )
    def pin_pos(pos_hbm, offx_hbm, offy_hbm, p2n_hbm, out_hbm,
                table_v, idx0, idx1, off0, off1, res0, res1,
                sem_t, sin0, sin1, sout0, sout1):
        wid = lax.axis_index("s") * 2 + lax.axis_index("c")
        base0 = wid * pins_per_tile
        slots = ((idx0, off0, res0, sin0, sout0),
                 (idx1, off1, res1, sin1, sout1))

        def one_pass(table_base, off_hbm, out_base):
            tcp = pltpu.async_copy(
                pos_hbm.at[pl.ds(table_base, n_nodes)], table_v, sem_t)

            def start_in(j, idxb, offb, sib):
                b = base0 + j * chunk
                pltpu.async_copy(p2n_hbm.at[pl.ds(b, chunk)], idxb, sib)
                pltpu.async_copy(off_hbm.at[pl.ds(b, chunk)], offb, sib)

            def wait_in(j, idxb, offb, sib):
                b = base0 + j * chunk
                pltpu.make_async_copy(
                    p2n_hbm.at[pl.ds(b, chunk)], idxb, sib).wait()
                pltpu.make_async_copy(
                    off_hbm.at[pl.ds(b, chunk)], offb, sib).wait()

            start_in(0, idx0, off0, sin0)
            start_in(1, idx1, off1, sin1)
            tcp.wait()

            def body(g, _):
                for b_i, (idxb, offb, resb, sib, sob) in enumerate(slots):
                    j = g * 2 + b_i
                    wait_in(j, idxb, offb, sib)

                    @pl.when(g > 0)
                    def _():
                        pltpu.make_async_copy(
                            resb,
                            out_hbm.at[pl.ds(out_base + base0 + (j - 2) * chunk,
                                             chunk)],
                            sob).wait()

                    def vec_body(k, _):
                        sl = pl.ds(k * _LANES, _LANES)
                        resb[sl] = offb[sl] + plsc.load_gather(
                            table_v, [idxb[sl]])
                        return 0

                    lax.fori_loop(0, chunk // _LANES, vec_body, 0, unroll=8)

                    pltpu.async_copy(
                        resb,
                        out_hbm.at[pl.ds(out_base + base0 + j * chunk, chunk)],
                        sob)

                    @pl.when(g < half - 1)
                    def _():
                        start_in(j + 2, idxb, offb, sib)
                return 0

            lax.fori_loop(0, half, body, 0)

            for b_i, (idxb, offb, resb, sib, sob) in enumerate(slots):
                j = num_chunks - 2 + b_i
                pltpu.make_async_copy(
                    resb,
                    out_hbm.at[pl.ds(out_base + base0 + j * chunk, chunk)],
                    sob).wait()

        one_pass(0, offx_hbm, 0)
        one_pass(n_nodes, offy_hbm, n_pins)

    return pin_pos


def kernel(pos, pin_offset_x, pin_offset_y, pin2node_map, flat_node2pin_map,
           flat_node2pin_start_map, num_physical_nodes):
    n_pins = pin2node_map.shape[0]
    n_nodes = pos.shape[0] // 2
    return _build(n_nodes, n_pins)(pos, pin_offset_x, pin_offset_y, pin2node_map)


# speedup vs baseline: 597.9824x; 1.5596x over previous
import functools

import jax
import jax.numpy as jnp
from jax import lax
from jax.experimental import pallas as pl
from jax.experimental.pallas import tpu as pltpu
from jax.experimental.pallas import tpu_sc as plsc

_LANES = 16
_NUM_WORKERS = 32


@functools.lru_cache(maxsize=None)
def _build(n_nodes, n_pins):
    pins_per_tile = n_pins // _NUM_WORKERS
    chunk = 2000
    num_chunks = pins_per_tile // chunk
    assert pins_per_tile % chunk == 0 and chunk % _LANES == 0
    assert num_chunks % 2 == 0
    half = num_chunks // 2

    mesh = plsc.VectorSubcoreMesh(core_axis_name="c", subcore_axis_name="s")

    @functools.partial(
        pl.kernel,
        mesh=mesh,
        out_type=jax.ShapeDtypeStruct((2 * n_pins,), jnp.float32),
        compiler_params=pltpu.CompilerParams(needs_layout_passes=False),
        scratch_types=[
            pltpu.VMEM((n_nodes,), jnp.float32),
            pltpu.VMEM((chunk,), jnp.int32),
            pltpu.VMEM((chunk,), jnp.int32),
            pltpu.VMEM((chunk,), jnp.float32),
            pltpu.VMEM((chunk,), jnp.float32),
            pltpu.VMEM((chunk,), jnp.float32),
            pltpu.VMEM((chunk,), jnp.float32),
            pltpu.SemaphoreType.DMA,
            pltpu.SemaphoreType.DMA,
            pltpu.SemaphoreType.DMA,
            pltpu.SemaphoreType.DMA,
            pltpu.SemaphoreType.DMA,
        ],
    )
    def pin_pos(pos_hbm, offx_hbm, offy_hbm, p2n_hbm, out_hbm,
                table_v, idx0, idx1, off0, off1, res0, res1,
                sem_t, sin0, sin1, sout0, sout1):
        wid = lax.axis_index("s") * 2 + lax.axis_index("c")
        base0 = wid * pins_per_tile
        slots = ((idx0, off0, res0, sin0, sout0),
                 (idx1, off1, res1, sin1, sout1))

        def one_pass(table_base, off_hbm, out_base):
            tcp = pltpu.async_copy(
                pos_hbm.at[pl.ds(table_base, n_nodes)], table_v, sem_t)

            def start_in(j, idxb, offb, sib):
                b = base0 + j * chunk
                pltpu.async_copy(p2n_hbm.at[pl.ds(b, chunk)], idxb, sib)
                pltpu.async_copy(off_hbm.at[pl.ds(b, chunk)], offb, sib)

            def wait_in(j, idxb, offb, sib):
                b = base0 + j * chunk
                pltpu.make_async_copy(
                    p2n_hbm.at[pl.ds(b, chunk)], idxb, sib).wait()
                pltpu.make_async_copy(
                    off_hbm.at[pl.ds(b, chunk)], offb, sib).wait()

            start_in(0, idx0, off0, sin0)
            start_in(1, idx1, off1, sin1)
            tcp.wait()

            def body(g, _):
                for b_i, (idxb, offb, resb, sib, sob) in enumerate(slots):
                    j = g * 2 + b_i
                    wait_in(j, idxb, offb, sib)

                    @pl.when(g > 0)
                    def _():
                        pltpu.make_async_copy(
                            resb,
                            out_hbm.at[pl.ds(out_base + base0 + (j - 2) * chunk,
                                             chunk)],
                            sob).wait()

                    @plsc.parallel_loop(0, chunk, _LANES, unroll=8)
                    def _(i):
                        sl = pl.ds(i, _LANES)
                        resb[sl] = offb[sl] + plsc.load_gather(
                            table_v, [idxb[sl]])

                    pltpu.async_copy(
                        resb,
                        out_hbm.at[pl.ds(out_base + base0 + j * chunk, chunk)],
                        sob)

                    @pl.when(g < half - 1)
                    def _():
                        start_in(j + 2, idxb, offb, sib)
                return 0

            lax.fori_loop(0, half, body, 0)

            for b_i, (idxb, offb, resb, sib, sob) in enumerate(slots):
                j = num_chunks - 2 + b_i
                pltpu.make_async_copy(
                    resb,
                    out_hbm.at[pl.ds(out_base + base0 + j * chunk, chunk)],
                    sob).wait()

        one_pass(0, offx_hbm, 0)
        one_pass(n_nodes, offy_hbm, n_pins)

    return pin_pos


def kernel(pos, pin_offset_x, pin_offset_y, pin2node_map, flat_node2pin_map,
           flat_node2pin_start_map, num_physical_nodes):
    n_pins = pin2node_map.shape[0]
    n_nodes = pos.shape[0] // 2
    return _build(n_nodes, n_pins)(pos, pin_offset_x, pin_offset_y, pin2node_map)


# speedup vs baseline: 915.7953x; 1.5315x over previous
import functools

import jax
import jax.numpy as jnp
from jax import lax
from jax.experimental import pallas as pl
from jax.experimental.pallas import tpu as pltpu
from jax.experimental.pallas import tpu_sc as plsc

_LANES = 16
_NUM_SUBCORES = 16
_NBUF = 4


@functools.lru_cache(maxsize=None)
def _build(n_nodes, n_pins):
    pins_per_tile = n_pins // _NUM_SUBCORES
    chunk = 2000
    num_chunks = pins_per_tile // chunk
    assert pins_per_tile % chunk == 0 and chunk % _LANES == 0
    assert num_chunks % _NBUF == 0
    rounds = num_chunks // _NBUF

    mesh = plsc.VectorSubcoreMesh(core_axis_name="c", subcore_axis_name="s")

    scratch = [pltpu.VMEM((n_nodes,), jnp.float32)]
    for _ in range(_NBUF):
        scratch += [
            pltpu.VMEM((chunk,), jnp.int32),
            pltpu.VMEM((chunk,), jnp.float32),
            pltpu.VMEM((chunk,), jnp.float32),
        ]
    scratch += [pltpu.SemaphoreType.DMA] * (1 + 2 * _NBUF)

    @functools.partial(
        pl.kernel,
        mesh=mesh,
        out_type=jax.ShapeDtypeStruct((2 * n_pins,), jnp.float32),
        compiler_params=pltpu.CompilerParams(needs_layout_passes=False),
        scratch_types=scratch,
    )
    def pin_pos(pos_hbm, offx_hbm, offy_hbm, p2n_hbm, out_hbm, table_v, *rest):
        bufs = rest[:3 * _NBUF]
        sems = rest[3 * _NBUF:]
        sem_t = sems[0]
        slots = tuple(
            (bufs[3 * b], bufs[3 * b + 1], bufs[3 * b + 2],
             sems[1 + 2 * b], sems[2 + 2 * b])
            for b in range(_NBUF))

        cid = lax.axis_index("c")
        sid = lax.axis_index("s")
        base0 = sid * pins_per_tile

        def do_coord(table_base, off_hbm, out_base):
            tcp = pltpu.async_copy(
                pos_hbm.at[pl.ds(table_base, n_nodes)], table_v, sem_t)

            def start_in(j, idxb, offb, sib):
                b = base0 + j * chunk
                pltpu.async_copy(p2n_hbm.at[pl.ds(b, chunk)], idxb, sib)
                pltpu.async_copy(off_hbm.at[pl.ds(b, chunk)], offb, sib)

            def wait_in(j, idxb, offb, sib):
                b = base0 + j * chunk
                pltpu.make_async_copy(
                    p2n_hbm.at[pl.ds(b, chunk)], idxb, sib).wait()
                pltpu.make_async_copy(
                    off_hbm.at[pl.ds(b, chunk)], offb, sib).wait()

            for b_i, (idxb, offb, resb, sib, sob) in enumerate(slots):
                start_in(b_i, idxb, offb, sib)
            tcp.wait()

            def body(g, _):
                for b_i, (idxb, offb, resb, sib, sob) in enumerate(slots):
                    j = g * _NBUF + b_i
                    wait_in(j, idxb, offb, sib)

                    @pl.when(g > 0)
                    def _():
                        pltpu.make_async_copy(
                            resb,
                            out_hbm.at[pl.ds(
                                out_base + base0 + (j - _NBUF) * chunk, chunk)],
                            sob).wait()

                    @plsc.parallel_loop(0, chunk, _LANES, unroll=8)
                    def _(i):
                        sl = pl.ds(i, _LANES)
                        resb[sl] = offb[sl] + plsc.load_gather(
                            table_v, [idxb[sl]])

                    pltpu.async_copy(
                        resb,
                        out_hbm.at[pl.ds(out_base + base0 + j * chunk, chunk)],
                        sob)

                    @pl.when(g < rounds - 1)
                    def _():
                        start_in(j + _NBUF, idxb, offb, sib)
                return 0

            lax.fori_loop(0, rounds, body, 0)

            for b_i, (idxb, offb, resb, sib, sob) in enumerate(slots):
                j = num_chunks - _NBUF + b_i
                pltpu.make_async_copy(
                    resb,
                    out_hbm.at[pl.ds(out_base + base0 + j * chunk, chunk)],
                    sob).wait()

        @pl.when(cid == 0)
        def _():
            do_coord(0, offx_hbm, 0)

        @pl.when(cid == 1)
        def _():
            do_coord(n_nodes, offy_hbm, n_pins)

    return pin_pos


def kernel(pos, pin_offset_x, pin_offset_y, pin2node_map, flat_node2pin_map,
           flat_node2pin_start_map, num_physical_nodes):
    n_pins = pin2node_map.shape[0]
    n_nodes = pos.shape[0] // 2
    return _build(n_nodes, n_pins)(pos, pin_offset_x, pin_offset_y, pin2node_map)


# speedup vs baseline: 958.1651x; 1.0463x over previous
import functools

import jax
import jax.numpy as jnp
from jax import lax
from jax.experimental import pallas as pl
from jax.experimental.pallas import tpu as pltpu
from jax.experimental.pallas import tpu_sc as plsc

_LANES = 16
_NUM_SUBCORES = 16
_NBUF = 5


@functools.lru_cache(maxsize=None)
def _build(n_nodes, n_pins):
    pins_per_tile = n_pins // _NUM_SUBCORES
    chunk = 2000
    num_chunks = pins_per_tile // chunk
    assert pins_per_tile % chunk == 0 and chunk % _LANES == 0
    assert num_chunks % _NBUF == 0
    rounds = num_chunks // _NBUF

    mesh = plsc.VectorSubcoreMesh(core_axis_name="c", subcore_axis_name="s")

    scratch = [pltpu.VMEM((n_nodes,), jnp.float32)]
    for _ in range(_NBUF):
        scratch += [
            pltpu.VMEM((chunk,), jnp.int32),
            pltpu.VMEM((chunk,), jnp.float32),
            pltpu.VMEM((chunk,), jnp.float32),
        ]
    scratch += [pltpu.SemaphoreType.DMA] * (1 + 2 * _NBUF)

    @functools.partial(
        pl.kernel,
        mesh=mesh,
        out_type=jax.ShapeDtypeStruct((2 * n_pins,), jnp.float32),
        compiler_params=pltpu.CompilerParams(needs_layout_passes=False),
        scratch_types=scratch,
    )
    def pin_pos(pos_hbm, offx_hbm, offy_hbm, p2n_hbm, out_hbm, table_v, *rest):
        bufs = rest[:3 * _NBUF]
        sems = rest[3 * _NBUF:]
        sem_t = sems[0]
        slots = tuple(
            (bufs[3 * b], bufs[3 * b + 1], bufs[3 * b + 2],
             sems[1 + 2 * b], sems[2 + 2 * b])
            for b in range(_NBUF))

        cid = lax.axis_index("c")
        sid = lax.axis_index("s")
        base0 = sid * pins_per_tile

        def do_coord(table_base, off_hbm, out_base):
            tcp = pltpu.async_copy(
                pos_hbm.at[pl.ds(table_base, n_nodes)], table_v, sem_t)

            def start_in(j, idxb, offb, sib):
                b = base0 + j * chunk
                pltpu.async_copy(p2n_hbm.at[pl.ds(b, chunk)], idxb, sib)
                pltpu.async_copy(off_hbm.at[pl.ds(b, chunk)], offb, sib)

            def wait_in(j, idxb, offb, sib):
                b = base0 + j * chunk
                pltpu.make_async_copy(
                    p2n_hbm.at[pl.ds(b, chunk)], idxb, sib).wait()
                pltpu.make_async_copy(
                    off_hbm.at[pl.ds(b, chunk)], offb, sib).wait()

            for b_i, (idxb, offb, resb, sib, sob) in enumerate(slots):
                start_in(b_i, idxb, offb, sib)
            tcp.wait()

            def body(g, _):
                for b_i, (idxb, offb, resb, sib, sob) in enumerate(slots):
                    j = g * _NBUF + b_i
                    wait_in(j, idxb, offb, sib)

                    @pl.when(g > 0)
                    def _():
                        pltpu.make_async_copy(
                            resb,
                            out_hbm.at[pl.ds(
                                out_base + base0 + (j - _NBUF) * chunk, chunk)],
                            sob).wait()

                    @plsc.parallel_loop(0, chunk, _LANES, unroll=8)
                    def _(i):
                        sl = pl.ds(i, _LANES)
                        resb[sl] = offb[sl] + plsc.load_gather(
                            table_v, [idxb[sl]])

                    pltpu.async_copy(
                        resb,
                        out_hbm.at[pl.ds(out_base + base0 + j * chunk, chunk)],
                        sob)

                    @pl.when(g < rounds - 1)
                    def _():
                        start_in(j + _NBUF, idxb, offb, sib)
                return 0

            lax.fori_loop(0, rounds, body, 0)

            for b_i, (idxb, offb, resb, sib, sob) in enumerate(slots):
                j = num_chunks - _NBUF + b_i
                pltpu.make_async_copy(
                    resb,
                    out_hbm.at[pl.ds(out_base + base0 + j * chunk, chunk)],
                    sob).wait()

        @pl.when(cid == 0)
        def _():
            do_coord(0, offx_hbm, 0)

        @pl.when(cid == 1)
        def _():
            do_coord(n_nodes, offy_hbm, n_pins)

    return pin_pos


def kernel(pos, pin_offset_x, pin_offset_y, pin2node_map, flat_node2pin_map,
           flat_node2pin_start_map, num_physical_nodes):
    n_pins = pin2node_map.shape[0]
    n_nodes = pos.shape[0] // 2
    return _build(n_nodes, n_pins)(pos, pin_offset_x, pin_offset_y, pin2node_map)
